# Initial kernel scaffold; baseline (speedup 1.0000x reference)
import jax
import jax.numpy as jnp
from jax.experimental import pallas as pl


def kernel(bbox_pred, conf_pred, anchors, gt_boxes):
    raise NotImplementedError("write your pallas kernel here")


# speedup vs baseline: 8.7829x; 8.7829x over previous
import jax
import jax.numpy as jnp
from jax.experimental import pallas as pl
from jax.experimental.pallas import tpu as pltpu

IOU_THRESHOLD = 0.5
NEG_POS_RATIO = 3
LOC_LOSS_WEIGHT = 1.0
ALPHA = 0.25
GAMMA = 2.0
EPS = 1e-7

B = 16
A_REAL = 20000
APAD = 20480
G = 32
CH = 2048
NCH = APAD // CH


def _loss_kernel(bbox_ref, conf_ref, anch_ref, gt_ref, out_ref,
                 dl_s, pos_s, acc_ref):
    i = pl.program_id(0)

    @pl.when(i == 0)
    def _init():
        acc_ref[0] = 0.0
        acc_ref[1] = 0.0
        acc_ref[2] = 0.0

    g = gt_ref[0]
    gx1 = g[:, 0:1]
    gy1 = g[:, 1:2]
    gx2 = g[:, 2:3]
    gy2 = g[:, 3:4]
    area_g = (gx2 - gx1) * (gy2 - gy1)

    row_iota = jax.lax.broadcasted_iota(jnp.int32, (G, CH), 0)

    cmax = jnp.full((G, 1), -1.0, dtype=jnp.float32)
    cidx = jnp.zeros((G, 1), dtype=jnp.int32)
    for j in range(NCH):
        base = j * CH
        ax1 = anch_ref[0:1, base:base + CH]
        ay1 = anch_ref[1:2, base:base + CH]
        ax2 = anch_ref[2:3, base:base + CH]
        ay2 = anch_ref[3:4, base:base + CH]
        px1 = bbox_ref[0, 0:1, base:base + CH]
        py1 = bbox_ref[0, 1:2, base:base + CH]
        px2 = bbox_ref[0, 2:3, base:base + CH]
        py2 = bbox_ref[0, 3:4, base:base + CH]

        ltx = jnp.maximum(ax1, gx1)
        lty = jnp.maximum(ay1, gy1)
        rbx = jnp.minimum(ax2, gx2)
        rby = jnp.minimum(ay2, gy2)
        inter = jnp.maximum(rbx - ltx, 0.0) * jnp.maximum(rby - lty, 0.0)
        area_a = (ax2 - ax1) * (ay2 - ay1)
        iou = inter / (area_a + area_g - inter + EPS)

        bgi = jnp.max(iou, axis=0, keepdims=True)
        bidx = jnp.min(jnp.where(iou == bgi, row_iota, G),
                       axis=0, keepdims=True)

        cm_j = jnp.max(iou, axis=1, keepdims=True)
        lane = jax.lax.broadcasted_iota(jnp.int32, (G, CH), 1) + base
        ci_j = jnp.min(jnp.where(iou == cm_j, lane, APAD),
                       axis=1, keepdims=True)
        upd = cm_j > cmax
        cidx = jnp.where(upd, ci_j, cidx)
        cmax = jnp.where(upd, cm_j, cmax)

        onehot = (row_iota == bidx).astype(jnp.float32)
        mx1 = jnp.sum(onehot * gx1, axis=0, keepdims=True)
        my1 = jnp.sum(onehot * gy1, axis=0, keepdims=True)
        mx2 = jnp.sum(onehot * gx2, axis=0, keepdims=True)
        my2 = jnp.sum(onehot * gy2, axis=0, keepdims=True)

        ix1 = jnp.maximum(px1, mx1)
        iy1 = jnp.maximum(py1, my1)
        ix2 = jnp.minimum(px2, mx2)
        iy2 = jnp.minimum(py2, my2)
        inter_d = jnp.maximum(ix2 - ix1, 0.0) * jnp.maximum(iy2 - iy1, 0.0)
        area_p = (px2 - px1) * (py2 - py1)
        area_t = (mx2 - mx1) * (my2 - my1)
        union = area_p + area_t - inter_d + EPS
        iou_d = inter_d / union
        cxp = (px1 + px2) * 0.5
        cyp = (py1 + py2) * 0.5
        cxt = (mx1 + mx2) * 0.5
        cyt = (my1 + my2) * 0.5
        center_dist = (cxp - cxt) ** 2 + (cyp - cyt) ** 2
        ex1 = jnp.minimum(px1, mx1)
        ey1 = jnp.minimum(py1, my1)
        ex2 = jnp.maximum(px2, mx2)
        ey2 = jnp.maximum(py2, my2)
        diag = (ex2 - ex1) ** 2 + (ey2 - ey1) ** 2 + EPS
        dl = 1.0 - iou_d + center_dist / diag

        dl_s[0:1, base:base + CH] = dl
        pos_s[0:1, base:base + CH] = (bgi > IOU_THRESHOLD).astype(jnp.float32)

    for j in range(NCH):
        base = j * CH
        lane = jax.lax.broadcasted_iota(jnp.int32, (G, CH), 1) + base
        forced = jnp.max((lane == cidx).astype(jnp.float32),
                         axis=0, keepdims=True)
        pos_s[0:1, base:base + CH] = jnp.maximum(
            pos_s[0:1, base:base + CH], forced)

    posf = pos_s[0:1, :]
    num_pos = jnp.sum(posf)

    x = conf_ref[0]
    p = jax.nn.sigmoid(x)
    ce = jnp.maximum(x, 0.0) - x * posf + jnp.log1p(jnp.exp(-jnp.abs(x)))
    p_t = p * posf + (1.0 - p) * (1.0 - posf)
    alpha_t = ALPHA * posf + (1.0 - ALPHA) * (1.0 - posf)
    om = 1.0 - p_t
    acl = alpha_t * om * om * ce

    pos_loss = jnp.sum(acl * posf)
    loc_i = jnp.sum(dl_s[0:1, :] * posf)

    lane_full = jax.lax.broadcasted_iota(jnp.int32, (1, APAD), 1)
    is_neg = jnp.logical_and(posf == 0.0, lane_full < A_REAL)
    nv = jnp.where(is_neg, acl, -1.0)
    bits = pltpu.bitcast(nv, jnp.int32)

    np_i = num_pos.astype(jnp.int32)
    k = jnp.minimum(np_i * NEG_POS_RATIO, A_REAL - np_i)

    def bs_body(_, lohi):
        lo, hi = lohi
        mid = lo + (hi - lo + 1) // 2
        cnt = jnp.sum((bits >= mid).astype(jnp.int32))
        good = cnt >= k
        return (jnp.where(good, mid, lo), jnp.where(good, hi, mid - 1))

    lo, _ = jax.lax.fori_loop(
        0, 31, bs_body, (jnp.int32(0), jnp.int32(0x7F7FFFFF)))

    gt_mask = bits > lo
    cnt_gt = jnp.sum(gt_mask.astype(jnp.int32))
    sum_gt = jnp.sum(jnp.where(gt_mask, nv, 0.0))
    tval = jnp.max(jnp.where(bits == lo, nv, -1.0))
    hard_neg = sum_gt + (k - cnt_gt).astype(jnp.float32) * tval

    kf = k.astype(jnp.float32)
    conf_i = jnp.where(
        k > 0,
        (pos_loss + hard_neg) / (num_pos + kf),
        pos_loss / jnp.maximum(num_pos, 1.0),
    )

    acc_ref[0] = acc_ref[0] + loc_i * LOC_LOSS_WEIGHT
    acc_ref[1] = acc_ref[1] + conf_i
    acc_ref[2] = acc_ref[2] + num_pos

    tl = acc_ref[0] / jnp.maximum(acc_ref[2], 1.0)
    tc = acc_ref[1] / B
    lane128 = jax.lax.broadcasted_iota(jnp.int32, (1, 128), 1)
    out_ref[...] = jnp.where(
        lane128 == 0, tl + tc,
        jnp.where(lane128 == 1, tc, jnp.where(lane128 == 2, tl, 0.0)))


def _run(bb, cf, an, gt):
    return pl.pallas_call(
        _loss_kernel,
        grid=(B,),
        in_specs=[
            pl.BlockSpec((1, 4, APAD), lambda i: (i, 0, 0)),
            pl.BlockSpec((1, 1, APAD), lambda i: (i, 0, 0)),
            pl.BlockSpec((4, APAD), lambda i: (0, 0)),
            pl.BlockSpec((1, G, 4), lambda i: (i, 0, 0)),
        ],
        out_specs=pl.BlockSpec((1, 128), lambda i: (0, 0)),
        out_shape=jax.ShapeDtypeStruct((1, 128), jnp.float32),
        scratch_shapes=[
            pltpu.VMEM((1, APAD), jnp.float32),
            pltpu.VMEM((1, APAD), jnp.float32),
            pltpu.SMEM((3,), jnp.float32),
        ],
        compiler_params=pltpu.CompilerParams(
            dimension_semantics=("arbitrary",)),
    )(bb, cf, an, gt)


def kernel(bbox_pred, conf_pred, anchors, gt_boxes):
    pad = APAD - A_REAL
    bb = jnp.pad(jnp.moveaxis(bbox_pred, 2, 1), ((0, 0), (0, 0), (0, pad)))
    cf = jnp.pad(conf_pred, ((0, 0), (0, pad)))[:, None, :]
    an = jnp.pad(anchors.T, ((0, 0), (0, pad)))
    out = _run(bb, cf, an, gt_boxes)
    return out[0, 0], out[0, 1], out[0, 2]
